# Initial kernel scaffold; baseline (speedup 1.0000x reference)
import jax
import jax.numpy as jnp
from jax.experimental import pallas as pl


def kernel(X, W, b):
    raise NotImplementedError("write your pallas kernel here")


# speedup vs baseline: 12.3580x; 12.3580x over previous
import functools

import jax
import jax.numpy as jnp
from jax import lax
from jax.experimental import pallas as pl
from jax.experimental.pallas import tpu as pltpu
from jax.experimental.pallas import tpu_sc as plsc

_NUM_SC = 2
_NUM_SUBCORES = 16
_NUM_WORKERS = _NUM_SC * _NUM_SUBCORES
_CHUNK = 128


def _mm_body(x_ref, wb_ref, bias_ref, o_ref):
    o_ref[...] = (
        jnp.dot(x_ref[...], wb_ref[...], preferred_element_type=jnp.float32)
        + bias_ref[...]
    )


def _linear_packed(x4, wb, bias4, block_rows=2048):
    rows = x4.shape[0]
    grid = rows // block_rows
    return pl.pallas_call(
        _mm_body,
        grid=(grid,),
        in_specs=[
            pl.BlockSpec((block_rows, 128), lambda i: (i, 0)),
            pl.BlockSpec((128, 128), lambda i: (0, 0)),
            pl.BlockSpec((1, 128), lambda i: (0, 0)),
        ],
        out_specs=pl.BlockSpec((block_rows, 128), lambda i: (i, 0)),
        out_shape=jax.ShapeDtypeStruct((rows, 128), jnp.float32),
    )(x4, wb, bias4)


def _sort_body(kt_ref, gidx_ref):
    kk = kt_ref[...]
    n = kk.shape[0]
    ii = lax.broadcasted_iota(jnp.int32, kk.shape, 0)
    icol = lax.broadcasted_iota(jnp.int32, (n, 1), 0)
    nbits = n.bit_length() - 1

    def outer(kb, carry):
        kk, ii = carry
        k = jnp.left_shift(1, kb)
        asc = (icol & k) == 0

        def inner(t, carry2):
            kk, ii = carry2
            j = jnp.left_shift(1, kb - 1 - t)
            bit = (icol & j) != 0
            kp = jnp.where(bit, pltpu.roll(kk, j, 0), pltpu.roll(kk, n - j, 0))
            ip = jnp.where(bit, pltpu.roll(ii, j, 0), pltpu.roll(ii, n - j, 0))
            gt = (kk > kp) | ((kk == kp) & (ii > ip))
            keep_max = bit == asc
            take = gt ^ keep_max
            return jnp.where(take, kp, kk), jnp.where(take, ip, ii)

        return lax.fori_loop(0, kb, inner, (kk, ii))

    kk, ii = lax.fori_loop(1, nbits + 1, outer, (kk, ii))
    lane = lax.broadcasted_iota(jnp.int32, kk.shape, 1)
    gidx_ref[...] = ii + lane * n


def _argsort_lanes(kt):
    n, l = kt.shape
    return pl.pallas_call(
        _sort_body,
        out_shape=jax.ShapeDtypeStruct((n, l), jnp.int32),
        compiler_params=pltpu.CompilerParams(vmem_limit_bytes=100 * 1024 * 1024),
    )(kt)


def _make_gather(rows_total, d_out):
    rows_per_w = rows_total // _NUM_WORKERS
    n_chunks = rows_per_w // _CHUNK
    mesh = plsc.VectorSubcoreMesh(core_axis_name="c", subcore_axis_name="s")

    @functools.partial(
        pl.kernel,
        out_type=jax.ShapeDtypeStruct((rows_total, d_out), jnp.float32),
        mesh=mesh,
        scratch_types=[
            pltpu.VMEM((n_chunks, _CHUNK), jnp.int32),
            pltpu.VMEM((_CHUNK, d_out), jnp.float32),
            pltpu.SemaphoreType.DMA,
        ],
        compiler_params=pltpu.CompilerParams(use_tc_tiling_on_sc=False),
    )
    def gather(table_hbm, idx_hbm, out_hbm, idx_v, rows_v, sem):
        wid = lax.axis_index("s") * _NUM_SC + lax.axis_index("c")
        pltpu.sync_copy(idx_hbm.at[wid], idx_v)
        base = wid * rows_per_w

        def chunk(ch, carry):
            pltpu.async_copy(table_hbm.at[idx_v.at[ch]], rows_v, sem).wait()
            pltpu.sync_copy(rows_v, out_hbm.at[pl.ds(base + ch * _CHUNK, _CHUNK)])
            return carry

        lax.fori_loop(0, n_chunks, chunk, 0)

    return gather


def kernel(X, W, b):
    B, S, d_in = X.shape
    d_out = W.shape[0]
    pack = 128 // d_in
    rows_total = B * S

    wb = jnp.kron(jnp.eye(pack, dtype=W.dtype), W.T)
    bias4 = jnp.tile(b, pack)[None, :]
    x4 = X.reshape(rows_total // pack, 128)
    e4 = _linear_packed(x4, wb, bias4)
    table = e4.reshape(rows_total, d_out)

    kt = X[:, :, 0].T
    gidx = _argsort_lanes(kt)

    rows_per_w = rows_total // _NUM_WORKERS
    idx3 = gidx.T.reshape(_NUM_WORKERS, rows_per_w // _CHUNK, _CHUNK)

    out = _make_gather(rows_total, d_out)(table, idx3)
    return out.reshape(B, S, d_out)


# speedup vs baseline: 28.0736x; 2.2717x over previous
import functools

import jax
import jax.numpy as jnp
from jax import lax
from jax.experimental import pallas as pl
from jax.experimental.pallas import tpu as pltpu
from jax.experimental.pallas import tpu_sc as plsc

_NUM_SC = 2
_NUM_SUBCORES = 16
_NUM_WORKERS = _NUM_SC * _NUM_SUBCORES
_CHUNK = 128


def _mm_body(x_ref, wb_ref, bias_ref, o_ref):
    o_ref[...] = (
        jnp.dot(x_ref[...], wb_ref[...], preferred_element_type=jnp.float32)
        + bias_ref[...]
    )


def _linear_packed(x4, wb, bias4, block_rows=2048):
    rows = x4.shape[0]
    grid = rows // block_rows
    return pl.pallas_call(
        _mm_body,
        grid=(grid,),
        in_specs=[
            pl.BlockSpec((block_rows, 128), lambda i: (i, 0)),
            pl.BlockSpec((128, 128), lambda i: (0, 0)),
            pl.BlockSpec((1, 128), lambda i: (0, 0)),
        ],
        out_specs=pl.BlockSpec((block_rows, 128), lambda i: (i, 0)),
        out_shape=jax.ShapeDtypeStruct((rows, 128), jnp.float32),
    )(x4, wb, bias4)


def _sort_body(kt_ref, gidx_ref, kk_ref, ii_ref):
    n, l = kt_ref.shape
    nbits = n.bit_length() - 1
    c = min(512, n)
    cbits = c.bit_length() - 1
    nch = n // c
    kk_ref[...] = kt_ref[...]
    ii_ref[...] = lax.broadcasted_iota(jnp.int32, (n, l), 0)
    icol = lax.broadcasted_iota(jnp.int32, (c, 1), 0)

    def cex_roll(kk, ii, asc, j):
        bit = (icol & j) != 0
        kp = jnp.where(bit, jnp.roll(kk, j, axis=0), jnp.roll(kk, -j, axis=0))
        ip = jnp.where(bit, jnp.roll(ii, j, axis=0), jnp.roll(ii, -j, axis=0))
        gt = (kk > kp) | ((kk == kp) & (ii > ip))
        take = gt ^ (bit == asc)
        return jnp.where(take, kp, kk), jnp.where(take, ip, ii)

    def local_sort_body(ch, carry):
        base = ch * c
        kk = kk_ref[pl.ds(base, c), :]
        ii = ii_ref[pl.ds(base, c), :]
        icg = icol + base
        for kb in range(1, cbits + 1):
            asc = (icg & (1 << kb)) == 0
            for jb in range(kb - 1, -1, -1):
                kk, ii = cex_roll(kk, ii, asc, 1 << jb)
        kk_ref[pl.ds(base, c), :] = kk
        ii_ref[pl.ds(base, c), :] = ii
        return carry

    lax.fori_loop(0, nch, local_sort_body, 0)

    for kb in range(cbits + 1, nbits + 1):
        k = 1 << kb
        for jb in range(kb - 1, cbits - 1, -1):
            j = 1 << jb
            ppb = j // c

            def pair_body(q, carry, j=j, k=k, ppb=ppb):
                base = (q // ppb) * 2 * j + (q % ppb) * c
                asc = (base & k) == 0
                ka = kk_ref[pl.ds(base, c), :]
                kb2 = kk_ref[pl.ds(base + j, c), :]
                ia = ii_ref[pl.ds(base, c), :]
                ib = ii_ref[pl.ds(base + j, c), :]
                gt = (ka > kb2) | ((ka == kb2) & (ia > ib))
                swap = gt == asc
                kk_ref[pl.ds(base, c), :] = jnp.where(swap, kb2, ka)
                kk_ref[pl.ds(base + j, c), :] = jnp.where(swap, ka, kb2)
                ii_ref[pl.ds(base, c), :] = jnp.where(swap, ib, ia)
                ii_ref[pl.ds(base + j, c), :] = jnp.where(swap, ia, ib)
                return carry

            lax.fori_loop(0, nch // 2, pair_body, 0)

        def merge_body(ch, carry, k=k):
            base = ch * c
            kk = kk_ref[pl.ds(base, c), :]
            ii = ii_ref[pl.ds(base, c), :]
            asc = ((icol + base) & k) == 0
            for jb in range(cbits - 1, -1, -1):
                kk, ii = cex_roll(kk, ii, asc, 1 << jb)
            kk_ref[pl.ds(base, c), :] = kk
            ii_ref[pl.ds(base, c), :] = ii
            return carry

        lax.fori_loop(0, nch, merge_body, 0)

    lane = lax.broadcasted_iota(jnp.int32, (n, l), 1)
    gidx_ref[...] = ii_ref[...] + lane * n


def _argsort_lanes(kt):
    n, l = kt.shape
    return pl.pallas_call(
        _sort_body,
        out_shape=jax.ShapeDtypeStruct((n, l), jnp.int32),
        scratch_shapes=[
            pltpu.VMEM((n, l), jnp.float32),
            pltpu.VMEM((n, l), jnp.int32),
        ],
        compiler_params=pltpu.CompilerParams(vmem_limit_bytes=100 * 1024 * 1024),
    )(kt)


def _make_gather(rows_total, d_out):
    rows_per_w = rows_total // _NUM_WORKERS
    n_chunks = rows_per_w // _CHUNK
    mesh = plsc.VectorSubcoreMesh(core_axis_name="c", subcore_axis_name="s")

    @functools.partial(
        pl.kernel,
        out_type=jax.ShapeDtypeStruct((rows_total, d_out), jnp.float32),
        mesh=mesh,
        scratch_types=[
            pltpu.VMEM((n_chunks, _CHUNK), jnp.int32),
            pltpu.VMEM((_CHUNK, d_out), jnp.float32),
            pltpu.SemaphoreType.DMA,
        ],
        compiler_params=pltpu.CompilerParams(use_tc_tiling_on_sc=False),
    )
    def gather(table_hbm, idx_hbm, out_hbm, idx_v, rows_v, sem):
        wid = lax.axis_index("s") * _NUM_SC + lax.axis_index("c")
        pltpu.sync_copy(idx_hbm.at[wid], idx_v)
        base = wid * rows_per_w

        def chunk(ch, carry):
            pltpu.async_copy(table_hbm.at[idx_v.at[ch]], rows_v, sem).wait()
            pltpu.sync_copy(rows_v, out_hbm.at[pl.ds(base + ch * _CHUNK, _CHUNK)])
            return carry

        lax.fori_loop(0, n_chunks, chunk, 0)

    return gather


def kernel(X, W, b):
    B, S, d_in = X.shape
    d_out = W.shape[0]
    pack = 128 // d_in
    rows_total = B * S

    wb = jnp.kron(jnp.eye(pack, dtype=W.dtype), W.T)
    bias4 = jnp.tile(b, pack)[None, :]
    x4 = X.reshape(rows_total // pack, 128)
    e4 = _linear_packed(x4, wb, bias4)
    table = e4.reshape(rows_total, d_out)

    kt = X[:, :, 0].T
    gidx = _argsort_lanes(kt)

    rows_per_w = rows_total // _NUM_WORKERS
    idx3 = gidx.T.reshape(_NUM_WORKERS, rows_per_w // _CHUNK, _CHUNK)

    out = _make_gather(rows_total, d_out)(table, idx3)
    return out.reshape(B, S, d_out)


# speedup vs baseline: 28.7859x; 1.0254x over previous
import functools

import jax
import jax.numpy as jnp
from jax import lax
from jax.experimental import pallas as pl
from jax.experimental.pallas import tpu as pltpu
from jax.experimental.pallas import tpu_sc as plsc

_NUM_SC = 2
_NUM_SUBCORES = 16
_NUM_WORKERS = _NUM_SC * _NUM_SUBCORES
_CHUNK = 128


def _mm_body(x_ref, wb_ref, bias_ref, o_ref):
    o_ref[...] = (
        jnp.dot(x_ref[...], wb_ref[...], preferred_element_type=jnp.float32)
        + bias_ref[...]
    )


def _linear_packed(x4, wb, bias4, block_rows=2048):
    rows = x4.shape[0]
    grid = rows // block_rows
    return pl.pallas_call(
        _mm_body,
        grid=(grid,),
        in_specs=[
            pl.BlockSpec((block_rows, 128), lambda i: (i, 0)),
            pl.BlockSpec((128, 128), lambda i: (0, 0)),
            pl.BlockSpec((1, 128), lambda i: (0, 0)),
        ],
        out_specs=pl.BlockSpec((block_rows, 128), lambda i: (i, 0)),
        out_shape=jax.ShapeDtypeStruct((rows, 128), jnp.float32),
    )(x4, wb, bias4)


def _sort_body(kt_ref, gidx_ref, kk_ref, ii_ref):
    n, l = kt_ref.shape
    nbits = n.bit_length() - 1
    c = min(128, n)
    cbits = c.bit_length() - 1
    nch = n // c
    kk_ref[...] = kt_ref[...]
    ii_ref[...] = lax.broadcasted_iota(jnp.int32, (n, l), 0)
    icol = lax.broadcasted_iota(jnp.int32, (c, 1), 0)

    def cex_roll(kk, ii, asc, j):
        bit = (icol & j) != 0
        kp = jnp.where(bit, jnp.roll(kk, j, axis=0), jnp.roll(kk, -j, axis=0))
        ip = jnp.where(bit, jnp.roll(ii, j, axis=0), jnp.roll(ii, -j, axis=0))
        gt = (kk > kp) | ((kk == kp) & (ii > ip))
        take = gt ^ (bit == asc)
        return jnp.where(take, kp, kk), jnp.where(take, ip, ii)

    def local_sort_body(ch, carry):
        base = ch * c
        kk = kk_ref[pl.ds(base, c), :]
        ii = ii_ref[pl.ds(base, c), :]
        icg = icol + base
        for kb in range(1, cbits + 1):
            asc = (icg & (1 << kb)) == 0
            for jb in range(kb - 1, -1, -1):
                kk, ii = cex_roll(kk, ii, asc, 1 << jb)
        kk_ref[pl.ds(base, c), :] = kk
        ii_ref[pl.ds(base, c), :] = ii
        return carry

    lax.fori_loop(0, nch, local_sort_body, 0)

    for kb in range(cbits + 1, nbits + 1):
        k = 1 << kb
        for jb in range(kb - 1, cbits - 1, -1):
            j = 1 << jb
            ppb = j // c

            def pair_body(q, carry, j=j, k=k, ppb=ppb):
                base = (q // ppb) * 2 * j + (q % ppb) * c
                asc = (base & k) == 0
                ka = kk_ref[pl.ds(base, c), :]
                kb2 = kk_ref[pl.ds(base + j, c), :]
                ia = ii_ref[pl.ds(base, c), :]
                ib = ii_ref[pl.ds(base + j, c), :]
                gt = (ka > kb2) | ((ka == kb2) & (ia > ib))
                swap = gt == asc
                kk_ref[pl.ds(base, c), :] = jnp.where(swap, kb2, ka)
                kk_ref[pl.ds(base + j, c), :] = jnp.where(swap, ka, kb2)
                ii_ref[pl.ds(base, c), :] = jnp.where(swap, ib, ia)
                ii_ref[pl.ds(base + j, c), :] = jnp.where(swap, ia, ib)
                return carry

            lax.fori_loop(0, nch // 2, pair_body, 0)

        def merge_body(ch, carry, k=k):
            base = ch * c
            kk = kk_ref[pl.ds(base, c), :]
            ii = ii_ref[pl.ds(base, c), :]
            asc = ((icol + base) & k) == 0
            for jb in range(cbits - 1, -1, -1):
                kk, ii = cex_roll(kk, ii, asc, 1 << jb)
            kk_ref[pl.ds(base, c), :] = kk
            ii_ref[pl.ds(base, c), :] = ii
            return carry

        lax.fori_loop(0, nch, merge_body, 0)

    lane = lax.broadcasted_iota(jnp.int32, (n, l), 1)
    gidx_ref[...] = ii_ref[...] + lane * n


def _argsort_lanes(kt):
    n, l = kt.shape
    return pl.pallas_call(
        _sort_body,
        out_shape=jax.ShapeDtypeStruct((n, l), jnp.int32),
        scratch_shapes=[
            pltpu.VMEM((n, l), jnp.float32),
            pltpu.VMEM((n, l), jnp.int32),
        ],
        compiler_params=pltpu.CompilerParams(vmem_limit_bytes=100 * 1024 * 1024),
    )(kt)


def _make_gather(rows_total, d_out):
    rows_per_w = rows_total // _NUM_WORKERS
    n_chunks = rows_per_w // _CHUNK
    mesh = plsc.VectorSubcoreMesh(core_axis_name="c", subcore_axis_name="s")

    @functools.partial(
        pl.kernel,
        out_type=jax.ShapeDtypeStruct((rows_total, d_out), jnp.float32),
        mesh=mesh,
        scratch_types=[
            pltpu.VMEM((n_chunks, _CHUNK), jnp.int32),
            pltpu.VMEM((_CHUNK, d_out), jnp.float32),
            pltpu.SemaphoreType.DMA,
        ],
        compiler_params=pltpu.CompilerParams(use_tc_tiling_on_sc=False),
    )
    def gather(table_hbm, idx_hbm, out_hbm, idx_v, rows_v, sem):
        wid = lax.axis_index("s") * _NUM_SC + lax.axis_index("c")
        pltpu.sync_copy(idx_hbm.at[wid], idx_v)
        base = wid * rows_per_w

        def chunk(ch, carry):
            pltpu.async_copy(table_hbm.at[idx_v.at[ch]], rows_v, sem).wait()
            pltpu.sync_copy(rows_v, out_hbm.at[pl.ds(base + ch * _CHUNK, _CHUNK)])
            return carry

        lax.fori_loop(0, n_chunks, chunk, 0)

    return gather


def kernel(X, W, b):
    B, S, d_in = X.shape
    d_out = W.shape[0]
    pack = 128 // d_in
    rows_total = B * S

    wb = jnp.kron(jnp.eye(pack, dtype=W.dtype), W.T)
    bias4 = jnp.tile(b, pack)[None, :]
    x4 = X.reshape(rows_total // pack, 128)
    e4 = _linear_packed(x4, wb, bias4)
    table = e4.reshape(rows_total, d_out)

    kt = X[:, :, 0].T
    gidx = _argsort_lanes(kt)

    rows_per_w = rows_total // _NUM_WORKERS
    idx3 = gidx.T.reshape(_NUM_WORKERS, rows_per_w // _CHUNK, _CHUNK)

    out = _make_gather(rows_total, d_out)(table, idx3)
    return out.reshape(B, S, d_out)


# speedup vs baseline: 31.1431x; 1.0819x over previous
import functools

import jax
import jax.numpy as jnp
from jax import lax
from jax.experimental import pallas as pl
from jax.experimental.pallas import tpu as pltpu
from jax.experimental.pallas import tpu_sc as plsc

_NUM_SC = 2
_NUM_SUBCORES = 16
_NUM_WORKERS = _NUM_SC * _NUM_SUBCORES
_CHUNK = 128


def _mm_body(x_ref, wb_ref, bias_ref, o_ref):
    o_ref[...] = (
        jnp.dot(x_ref[...], wb_ref[...], preferred_element_type=jnp.float32)
        + bias_ref[...]
    )


def _linear_packed(x4, wb, bias4, block_rows=2048):
    rows = x4.shape[0]
    grid = rows // block_rows
    return pl.pallas_call(
        _mm_body,
        grid=(grid,),
        in_specs=[
            pl.BlockSpec((block_rows, 128), lambda i: (i, 0)),
            pl.BlockSpec((128, 128), lambda i: (0, 0)),
            pl.BlockSpec((1, 128), lambda i: (0, 0)),
        ],
        out_specs=pl.BlockSpec((block_rows, 128), lambda i: (i, 0)),
        out_shape=jax.ShapeDtypeStruct((rows, 128), jnp.float32),
    )(x4, wb, bias4)


def _sort_body(kt_ref, gidx_ref, kk_ref, ii_ref):
    n, l = kt_ref.shape
    nbits = n.bit_length() - 1
    c = min(128, n)
    cbits = c.bit_length() - 1
    nch = n // c
    kk_ref[...] = kt_ref[...]
    ii_ref[...] = lax.broadcasted_iota(jnp.int32, (n, l), 0)
    icol = lax.broadcasted_iota(jnp.int32, (c, 1), 0)

    def cex_roll(kk, ii, asc, j):
        bit = (icol & j) != 0
        kp = jnp.where(bit, jnp.roll(kk, j, axis=0), jnp.roll(kk, -j, axis=0))
        ip = jnp.where(bit, jnp.roll(ii, j, axis=0), jnp.roll(ii, -j, axis=0))
        gt = (kk > kp) | ((kk == kp) & (ii > ip))
        take = gt ^ (bit == asc)
        return jnp.where(take, kp, kk), jnp.where(take, ip, ii)

    def local_sort_body(ch, carry):
        base = ch * c
        kk = kk_ref[pl.ds(base, c), :]
        ii = ii_ref[pl.ds(base, c), :]
        icg = icol + base
        for kb in range(1, cbits + 1):
            asc = (icg & (1 << kb)) == 0
            for jb in range(kb - 1, -1, -1):
                kk, ii = cex_roll(kk, ii, asc, 1 << jb)
        kk_ref[pl.ds(base, c), :] = kk
        ii_ref[pl.ds(base, c), :] = ii
        return carry

    lax.fori_loop(0, nch, local_sort_body, 0)

    for kb in range(cbits + 1, nbits + 1):
        k = 1 << kb
        for jb in range(kb - 1, cbits - 1, -1):
            j = 1 << jb
            ppb = j // c

            def pair_body(q, carry, j=j, k=k, ppb=ppb):
                base = (q // ppb) * 2 * j + (q % ppb) * c
                asc = (base & k) == 0
                ka = kk_ref[pl.ds(base, c), :]
                kb2 = kk_ref[pl.ds(base + j, c), :]
                ia = ii_ref[pl.ds(base, c), :]
                ib = ii_ref[pl.ds(base + j, c), :]
                gt = (ka > kb2) | ((ka == kb2) & (ia > ib))
                swap = gt == asc
                kk_ref[pl.ds(base, c), :] = jnp.where(swap, kb2, ka)
                kk_ref[pl.ds(base + j, c), :] = jnp.where(swap, ka, kb2)
                ii_ref[pl.ds(base, c), :] = jnp.where(swap, ib, ia)
                ii_ref[pl.ds(base + j, c), :] = jnp.where(swap, ia, ib)
                return carry

            lax.fori_loop(0, nch // 2, pair_body, 0)

        def merge_body(ch, carry, k=k):
            base = ch * c
            kk = kk_ref[pl.ds(base, c), :]
            ii = ii_ref[pl.ds(base, c), :]
            asc = ((icol + base) & k) == 0
            for jb in range(cbits - 1, -1, -1):
                kk, ii = cex_roll(kk, ii, asc, 1 << jb)
            kk_ref[pl.ds(base, c), :] = kk
            ii_ref[pl.ds(base, c), :] = ii
            return carry

        lax.fori_loop(0, nch, merge_body, 0)

    lane = lax.broadcasted_iota(jnp.int32, (n, l), 1)
    gidx_ref[...] = ii_ref[...] + lane * n


def _argsort_lanes(kt):
    n, l = kt.shape
    return pl.pallas_call(
        _sort_body,
        out_shape=jax.ShapeDtypeStruct((n, l), jnp.int32),
        scratch_shapes=[
            pltpu.VMEM((n, l), jnp.float32),
            pltpu.VMEM((n, l), jnp.int32),
        ],
        compiler_params=pltpu.CompilerParams(vmem_limit_bytes=100 * 1024 * 1024),
    )(kt)


def _make_gather(rows_total, d_out):
    rows_per_w = rows_total // _NUM_WORKERS
    n_chunks = rows_per_w // _CHUNK
    mesh = plsc.VectorSubcoreMesh(core_axis_name="c", subcore_axis_name="s")

    @functools.partial(
        pl.kernel,
        out_type=jax.ShapeDtypeStruct((rows_total, d_out), jnp.float32),
        mesh=mesh,
        scratch_types=[
            pltpu.VMEM((n_chunks, _CHUNK), jnp.int32),
            pltpu.VMEM((_CHUNK, d_out), jnp.float32),
            pltpu.VMEM((_CHUNK, d_out), jnp.float32),
            pltpu.SemaphoreType.DMA,
            pltpu.SemaphoreType.DMA,
            pltpu.SemaphoreType.DMA,
            pltpu.SemaphoreType.DMA,
        ],
        compiler_params=pltpu.CompilerParams(use_tc_tiling_on_sc=False),
    )
    def gather(table_hbm, idx_hbm, out_hbm, idx_v, rows0, rows1, g0, g1, s0, s1):
        wid = lax.axis_index("s") * _NUM_SC + lax.axis_index("c")
        pltpu.sync_copy(idx_hbm.at[wid], idx_v)
        base = wid * rows_per_w
        rows = (rows0, rows1)
        gs = (g0, g1)
        ss = (s0, s1)

        pltpu.async_copy(table_hbm.at[idx_v.at[0]], rows0, g0)

        def body(t, carry):
            for b in (0, 1):
                ch = 2 * t + b
                ob = 1 - b

                @pl.when(ch + 1 < n_chunks)
                def _():
                    @pl.when(ch >= 1)
                    def _():
                        pltpu.make_async_copy(
                            rows[ob],
                            out_hbm.at[pl.ds(base + (ch - 1) * _CHUNK, _CHUNK)],
                            ss[ob],
                        ).wait()

                    pltpu.async_copy(
                        table_hbm.at[idx_v.at[ch + 1]], rows[ob], gs[ob]
                    )

                pltpu.make_async_copy(
                    table_hbm.at[idx_v.at[ch]], rows[b], gs[b]
                ).wait()
                pltpu.async_copy(
                    rows[b], out_hbm.at[pl.ds(base + ch * _CHUNK, _CHUNK)], ss[b]
                )
            return carry

        lax.fori_loop(0, n_chunks // 2, body, 0)
        for b in (0, 1):
            pltpu.make_async_copy(
                rows[b],
                out_hbm.at[pl.ds(base + (n_chunks - 2 + b) * _CHUNK, _CHUNK)],
                ss[b],
            ).wait()

    return gather


def kernel(X, W, b):
    B, S, d_in = X.shape
    d_out = W.shape[0]
    pack = 128 // d_in
    rows_total = B * S

    wb = jnp.kron(jnp.eye(pack, dtype=W.dtype), W.T)
    bias4 = jnp.tile(b, pack)[None, :]
    x4 = X.reshape(rows_total // pack, 128)
    e4 = _linear_packed(x4, wb, bias4)
    table = e4.reshape(rows_total, d_out)

    kt = X[:, :, 0].T
    gidx = _argsort_lanes(kt)

    rows_per_w = rows_total // _NUM_WORKERS
    idx3 = gidx.T.reshape(_NUM_WORKERS, rows_per_w // _CHUNK, _CHUNK)

    out = _make_gather(rows_total, d_out)(table, idx3)
    return out.reshape(B, S, d_out)


# speedup vs baseline: 31.3037x; 1.0052x over previous
import functools

import jax
import jax.numpy as jnp
from jax import lax
from jax.experimental import pallas as pl
from jax.experimental.pallas import tpu as pltpu
from jax.experimental.pallas import tpu_sc as plsc

_NUM_SC = 2
_NUM_SUBCORES = 16
_NUM_WORKERS = _NUM_SC * _NUM_SUBCORES
_CHUNK = 128


def _mm_body(x_ref, wb_ref, bias_ref, o_ref):
    o_ref[...] = (
        jnp.dot(x_ref[...], wb_ref[...], preferred_element_type=jnp.float32)
        + bias_ref[...]
    )


def _linear_packed(x4, wb, bias4, block_rows=2048):
    rows = x4.shape[0]
    grid = rows // block_rows
    return pl.pallas_call(
        _mm_body,
        grid=(grid,),
        in_specs=[
            pl.BlockSpec((block_rows, 128), lambda i: (i, 0)),
            pl.BlockSpec((128, 128), lambda i: (0, 0)),
            pl.BlockSpec((1, 128), lambda i: (0, 0)),
        ],
        out_specs=pl.BlockSpec((block_rows, 128), lambda i: (i, 0)),
        out_shape=jax.ShapeDtypeStruct((rows, 128), jnp.float32),
    )(x4, wb, bias4)


def _sort_body(kt_ref, gidx_ref, kk_ref, ii_ref):
    n, l = kt_ref.shape
    nbits = n.bit_length() - 1
    c = min(128, n)
    cbits = c.bit_length() - 1
    nch = n // c
    kk_ref[...] = kt_ref[...]
    ii_ref[...] = lax.broadcasted_iota(jnp.int32, (n, l), 0)
    icol = lax.broadcasted_iota(jnp.int32, (c, 1), 0)

    def cex_roll(kk, ii, asc, j):
        bit = (icol & j) != 0
        kp = jnp.where(bit, jnp.roll(kk, j, axis=0), jnp.roll(kk, -j, axis=0))
        ip = jnp.where(bit, jnp.roll(ii, j, axis=0), jnp.roll(ii, -j, axis=0))
        gt = (kk > kp) | ((kk == kp) & (ii > ip))
        take = gt ^ (bit == asc)
        return jnp.where(take, kp, kk), jnp.where(take, ip, ii)

    def local_sort_body(ch, carry):
        base = ch * c
        kk = kk_ref[pl.ds(base, c), :]
        ii = ii_ref[pl.ds(base, c), :]
        icg = icol + base
        for kb in range(1, cbits + 1):
            asc = (icg & (1 << kb)) == 0
            for jb in range(kb - 1, -1, -1):
                kk, ii = cex_roll(kk, ii, asc, 1 << jb)
        kk_ref[pl.ds(base, c), :] = kk
        ii_ref[pl.ds(base, c), :] = ii
        return carry

    lax.fori_loop(0, nch, local_sort_body, 0)

    for kb in range(cbits + 1, nbits + 1):
        k = 1 << kb
        for jb in range(kb - 1, cbits - 1, -1):
            j = 1 << jb
            ppb = j // c

            def pair_body(q, carry, j=j, k=k, ppb=ppb):
                base = (q // ppb) * 2 * j + (q % ppb) * c
                asc = (base & k) == 0
                ka = kk_ref[pl.ds(base, c), :]
                kb2 = kk_ref[pl.ds(base + j, c), :]
                ia = ii_ref[pl.ds(base, c), :]
                ib = ii_ref[pl.ds(base + j, c), :]
                gt = (ka > kb2) | ((ka == kb2) & (ia > ib))
                swap = gt == asc
                kk_ref[pl.ds(base, c), :] = jnp.where(swap, kb2, ka)
                kk_ref[pl.ds(base + j, c), :] = jnp.where(swap, ka, kb2)
                ii_ref[pl.ds(base, c), :] = jnp.where(swap, ib, ia)
                ii_ref[pl.ds(base + j, c), :] = jnp.where(swap, ia, ib)
                return carry

            lax.fori_loop(0, nch // 2, pair_body, 0)

        def merge_body(ch, carry, k=k):
            base = ch * c
            kk = kk_ref[pl.ds(base, c), :]
            ii = ii_ref[pl.ds(base, c), :]
            asc = ((icol + base) & k) == 0
            for jb in range(cbits - 1, -1, -1):
                kk, ii = cex_roll(kk, ii, asc, 1 << jb)
            kk_ref[pl.ds(base, c), :] = kk
            ii_ref[pl.ds(base, c), :] = ii
            return carry

        lax.fori_loop(0, nch, merge_body, 0)

    tt = jnp.swapaxes(ii_ref[...], 0, 1)
    boff = lax.broadcasted_iota(jnp.int32, (l, n), 0) * n
    gidx_ref[...] = (tt + boff).reshape(n * l // 128, 128)


def _argsort_lanes(kt):
    n, l = kt.shape
    return pl.pallas_call(
        _sort_body,
        out_shape=jax.ShapeDtypeStruct((n * l // 128, 128), jnp.int32),
        scratch_shapes=[
            pltpu.VMEM((n, l), jnp.float32),
            pltpu.VMEM((n, l), jnp.int32),
        ],
        compiler_params=pltpu.CompilerParams(vmem_limit_bytes=100 * 1024 * 1024),
    )(kt)


def _make_gather(rows_total, d_out):
    rows_per_w = rows_total // _NUM_WORKERS
    n_chunks = rows_per_w // _CHUNK
    mesh = plsc.VectorSubcoreMesh(core_axis_name="c", subcore_axis_name="s")

    @functools.partial(
        pl.kernel,
        out_type=jax.ShapeDtypeStruct((rows_total, d_out), jnp.float32),
        mesh=mesh,
        scratch_types=[
            pltpu.VMEM((n_chunks, _CHUNK), jnp.int32),
            pltpu.VMEM((_CHUNK, d_out), jnp.float32),
            pltpu.VMEM((_CHUNK, d_out), jnp.float32),
            pltpu.SemaphoreType.DMA,
            pltpu.SemaphoreType.DMA,
            pltpu.SemaphoreType.DMA,
            pltpu.SemaphoreType.DMA,
        ],
        compiler_params=pltpu.CompilerParams(use_tc_tiling_on_sc=False),
    )
    def gather(table_hbm, idx_hbm, out_hbm, idx_v, rows0, rows1, g0, g1, s0, s1):
        wid = lax.axis_index("s") * _NUM_SC + lax.axis_index("c")
        pltpu.sync_copy(idx_hbm.at[wid], idx_v)
        base = wid * rows_per_w
        rows = (rows0, rows1)
        gs = (g0, g1)
        ss = (s0, s1)

        pltpu.async_copy(table_hbm.at[idx_v.at[0]], rows0, g0)

        def body(t, carry):
            for b in (0, 1):
                ch = 2 * t + b
                ob = 1 - b

                @pl.when(ch + 1 < n_chunks)
                def _():
                    @pl.when(ch >= 1)
                    def _():
                        pltpu.make_async_copy(
                            rows[ob],
                            out_hbm.at[pl.ds(base + (ch - 1) * _CHUNK, _CHUNK)],
                            ss[ob],
                        ).wait()

                    pltpu.async_copy(
                        table_hbm.at[idx_v.at[ch + 1]], rows[ob], gs[ob]
                    )

                pltpu.make_async_copy(
                    table_hbm.at[idx_v.at[ch]], rows[b], gs[b]
                ).wait()
                pltpu.async_copy(
                    rows[b], out_hbm.at[pl.ds(base + ch * _CHUNK, _CHUNK)], ss[b]
                )
            return carry

        lax.fori_loop(0, n_chunks // 2, body, 0)
        for b in (0, 1):
            pltpu.make_async_copy(
                rows[b],
                out_hbm.at[pl.ds(base + (n_chunks - 2 + b) * _CHUNK, _CHUNK)],
                ss[b],
            ).wait()

    return gather


def kernel(X, W, b):
    B, S, d_in = X.shape
    d_out = W.shape[0]
    pack = 128 // d_in
    rows_total = B * S

    wb = jnp.kron(jnp.eye(pack, dtype=W.dtype), W.T)
    bias4 = jnp.tile(b, pack)[None, :]
    x4 = X.reshape(rows_total // pack, 128)
    e4 = _linear_packed(x4, wb, bias4)
    table = e4.reshape(rows_total, d_out)

    kt = X[:, :, 0].T
    gidx = _argsort_lanes(kt)

    rows_per_w = rows_total // _NUM_WORKERS
    idx3 = gidx.reshape(_NUM_WORKERS, rows_per_w // _CHUNK, _CHUNK)

    out = _make_gather(rows_total, d_out)(table, idx3)
    return out.reshape(B, S, d_out)
